# baseline (speedup 1.0000x reference)
import jax
import jax.numpy as jnp
from jax import lax
from jax.experimental import pallas as pl
from jax.experimental.pallas import tpu as pltpu

D_IN = 2304
D_SAE = 18432
TOPK = 100
HALF = D_SAE // 2
BATCH = 1024
ROWS2 = 2 * BATCH

ENC_TILE = 768
DEC_TILE = 1024
SIM_TILE = 2304
K2_ROWS = 64
NPASS = 24

_PREC = lax.Precision.DEFAULT


def _k1_body(x_ref, w_ref, b_ref, out_ref):
    out_ref[...] = (
        jnp.dot(x_ref[...], w_ref[...], preferred_element_type=jnp.float32,
                precision=_PREC)
        + b_ref[...]
    )


def _encode_pre(x2, w_enc, b_enc2):
    grid = (D_SAE // ENC_TILE,)
    return pl.pallas_call(
        _k1_body,
        grid=grid,
        in_specs=[
            pl.BlockSpec((ROWS2, D_IN), lambda j: (0, 0)),
            pl.BlockSpec((D_IN, ENC_TILE), lambda j: (0, j)),
            pl.BlockSpec((1, ENC_TILE), lambda j: (0, j)),
        ],
        out_specs=pl.BlockSpec((ROWS2, ENC_TILE), lambda j: (0, j)),
        out_shape=jax.ShapeDtypeStruct((ROWS2, D_SAE), jnp.float32),
    )(x2, w_enc, b_enc2)


def _k2_body(pre_ref, thr_ref, nh2_ref):
    p = pre_ref[...]
    bits = lax.bitcast_convert_type(p, jnp.int32)
    bitsp = jnp.where(p > 0.0, bits, 0)
    k16 = ((bitsp >> 15) - 32768).astype(jnp.int16)
    tu = jnp.zeros((K2_ROWS, 1), jnp.int32)
    c = jnp.zeros((K2_ROWS, 1), jnp.int32)

    def _count16(mask16):
        m16 = mask16.astype(jnp.int16)
        s = m16[:, : D_SAE // 2] + m16[:, D_SAE // 2:]
        s = s[:, : D_SAE // 4] + s[:, D_SAE // 4:]
        s = s[:, : D_SAE // 8] + s[:, D_SAE // 8:]
        s = s[:, : D_SAE // 16] + s[:, D_SAE // 16:]
        return jnp.sum(s.astype(jnp.int32), axis=1, keepdims=True)

    for b in range(15, -1, -1):
        cand = tu + (1 << b)
        cand16 = (cand - 32768).astype(jnp.int16)
        cnt = _count16(k16 >= cand16)
        take = cnt >= TOPK
        tu = jnp.where(take, cand, tu)
        c = jnp.where(take, cnt, c)
    tu16 = (tu - 32768).astype(jnp.int16)
    c_gt = _count16(k16 > tu16)
    lo16 = ((bitsp & 0x7FFF) - 32768).astype(jnp.int16)
    bandlo = jnp.where(k16 == tu16, lo16, jnp.int16(-32768))
    tlo = jnp.zeros((K2_ROWS, 1), jnp.int32)
    for b in range(14, 30 - NPASS, -1):
        cand = tlo + (1 << b)
        cand16 = (cand - 32768).astype(jnp.int16)
        cnt = c_gt + _count16(bandlo >= cand16)
        take = cnt >= TOPK
        tlo = jnp.where(take, cand, tlo)
        c = jnp.where(take, cnt, c)
    t = (tu << 15) + tlo
    thr = jnp.maximum(t, 1)
    c = jnp.where(t > 0, c, 0)

    big = jnp.int32(0x7FFFFFFF)

    def _over(state):
        thr_, c_ = state
        return jnp.any(c_ > TOPK)

    def _drop_min(state):
        thr_, c_ = state
        act = bitsp >= thr_
        mk = jnp.min(jnp.where(act, bitsp, big), axis=1, keepdims=True)
        nmin = jnp.sum((act & (bitsp == mk)).astype(jnp.int32), axis=1,
                       keepdims=True)
        over = c_ > TOPK
        thr_ = jnp.where(over, mk + 1, thr_)
        c_ = jnp.where(over, c_ - nmin, c_)
        return thr_, c_

    thr, c = lax.while_loop(_over, _drop_min, (thr, c))
    thr_ref[...] = thr
    z = jnp.where(bitsp >= thr, p, 0.0)
    col = lax.broadcasted_iota(jnp.int32, (K2_ROWS, D_SAE), 1)
    zh = jnp.where(col < HALF, z, 0.0)
    nh2_ref[...] = jnp.sum(zh * zh, axis=1, keepdims=True)


def _topk_threshold(pre):
    grid = (ROWS2 // K2_ROWS,)
    return pl.pallas_call(
        _k2_body,
        grid=grid,
        in_specs=[pl.BlockSpec((K2_ROWS, D_SAE), lambda i: (i, 0))],
        out_specs=[
            pl.BlockSpec((K2_ROWS, 1), lambda i: (i, 0)),
            pl.BlockSpec((K2_ROWS, 1), lambda i: (i, 0)),
        ],
        out_shape=[
            jax.ShapeDtypeStruct((ROWS2, 1), jnp.int32),
            jax.ShapeDtypeStruct((ROWS2, 1), jnp.float32),
        ],
    )(pre)


def _mask_tile(p, thr):
    bits = lax.bitcast_convert_type(p, jnp.int32)
    return jnp.where(bits >= thr, p, 0.0)


def _k3_body(pre_ref, thr_ref, w_ref, xc_ref, bd_ref,
             xhat_ref, z_ref, lhi_ref, lfull_ref):
    c = pl.program_id(0)
    nhi = HALF // DEC_TILE
    denom = float(BATCH * D_IN)

    @pl.when(c == 0)
    def _init():
        xhat_ref[...] = jnp.zeros_like(xhat_ref)

    zt = _mask_tile(pre_ref[...], thr_ref[...])
    z_ref[...] = zt
    prod = jnp.dot(zt.astype(jnp.bfloat16), w_ref[...].astype(jnp.bfloat16),
                   preferred_element_type=jnp.float32)
    xhat_ref[...] += prod

    @pl.when(c == nhi - 1)
    def _hi():
        dh = (xhat_ref[...] + bd_ref[...]) - xc_ref[...]
        eh = jnp.sum(jnp.sum(dh * dh, axis=1, keepdims=True), axis=0,
                     keepdims=True)
        lhi_ref[...] = eh / denom

    @pl.when(c == (D_SAE // DEC_TILE) - 1)
    def _fin():
        full = xhat_ref[...] + bd_ref[...]
        xhat_ref[...] = full
        d = full - xc_ref[...]
        ef = jnp.sum(jnp.sum(d * d, axis=1, keepdims=True), axis=0,
                     keepdims=True)
        lfull_ref[...] = ef / denom


def _decode_losses(pre, thr, w_dec, x_cur, b_dec2):
    grid = (D_SAE // DEC_TILE,)
    return pl.pallas_call(
        _k3_body,
        grid=grid,
        in_specs=[
            pl.BlockSpec((BATCH, DEC_TILE), lambda c: (0, c)),
            pl.BlockSpec((BATCH, 1), lambda c: (0, 0)),
            pl.BlockSpec((DEC_TILE, D_IN), lambda c: (c, 0)),
            pl.BlockSpec((BATCH, D_IN), lambda c: (0, 0)),
            pl.BlockSpec((1, D_IN), lambda c: (0, 0)),
        ],
        out_specs=[
            pl.BlockSpec((BATCH, D_IN), lambda c: (0, 0)),
            pl.BlockSpec((BATCH, DEC_TILE), lambda c: (0, c)),
            pl.BlockSpec((1, 1), lambda c: (0, 0)),
            pl.BlockSpec((1, 1), lambda c: (0, 0)),
        ],
        out_shape=[
            jax.ShapeDtypeStruct((BATCH, D_IN), jnp.float32),
            jax.ShapeDtypeStruct((BATCH, D_SAE), jnp.float32),
            jax.ShapeDtypeStruct((1, 1), jnp.float32),
            jax.ShapeDtypeStruct((1, 1), jnp.float32),
        ],
    )(pre, thr, w_dec, x_cur, b_dec2)


def _k4_body(pa_ref, pb_ref, ta_ref, tb_ref, na2_ref, nb2_ref, out_ref,
             sim_ref):
    j = pl.program_id(0)

    @pl.when(j == 0)
    def _init():
        sim_ref[...] = jnp.zeros_like(sim_ref)

    na = jnp.maximum(jnp.sqrt(na2_ref[...]), 1e-8)
    nb = jnp.maximum(jnp.sqrt(nb2_ref[...]), 1e-8)
    za = _mask_tile(pa_ref[...], ta_ref[...])
    zb = _mask_tile(pb_ref[...], tb_ref[...])
    zan = (za / na).astype(jnp.bfloat16)
    zbn = (zb / nb).astype(jnp.bfloat16)
    sim_ref[...] += lax.dot_general(
        zan, zbn, (((1,), (1,)), ((), ())),
        preferred_element_type=jnp.float32)

    @pl.when(j == (HALF // SIM_TILE) - 1)
    def _fin():
        s = sim_ref[...]
        rmax = jnp.max(s, axis=1, keepdims=True)
        lse_r = rmax + jnp.log(jnp.sum(jnp.exp(s - rmax), axis=1,
                                       keepdims=True))
        cmax = jnp.max(s, axis=0, keepdims=True)
        lse_c = cmax + jnp.log(jnp.sum(jnp.exp(s - cmax), axis=0,
                                       keepdims=True))
        ii = lax.broadcasted_iota(jnp.int32, (BATCH, BATCH), 0)
        jj = lax.broadcasted_iota(jnp.int32, (BATCH, BATCH), 1)
        dsum = jnp.sum(jnp.sum(jnp.where(ii == jj, s, 0.0), axis=1,
                               keepdims=True), axis=0, keepdims=True)
        sr = jnp.sum(jnp.sum(lse_r, axis=1, keepdims=True), axis=0,
                     keepdims=True)
        sc = jnp.sum(jnp.sum(lse_c, axis=1, keepdims=True), axis=0,
                     keepdims=True)
        out_ref[...] = (-dsum + 0.5 * (sr + sc)) / float(BATCH)


def _info_nce_loss(pre, thr, nh2):
    grid = (HALF // SIM_TILE,)
    return pl.pallas_call(
        _k4_body,
        grid=grid,
        in_specs=[
            pl.BlockSpec((BATCH, SIM_TILE), lambda j: (0, j)),
            pl.BlockSpec((BATCH, SIM_TILE), lambda j: (1, j)),
            pl.BlockSpec((BATCH, 1), lambda j: (0, 0)),
            pl.BlockSpec((BATCH, 1), lambda j: (1, 0)),
            pl.BlockSpec((BATCH, 1), lambda j: (0, 0)),
            pl.BlockSpec((BATCH, 1), lambda j: (1, 0)),
        ],
        out_specs=pl.BlockSpec((1, 1), lambda j: (0, 0)),
        out_shape=jax.ShapeDtypeStruct((1, 1), jnp.float32),
        scratch_shapes=[pltpu.VMEM((BATCH, BATCH), jnp.float32)],
    )(pre, pre, thr, thr, nh2, nh2)


def kernel(x, W_enc, W_dec, b_enc, b_dec):
    x_cur = x[:, 1, :]
    x2 = jnp.concatenate([x_cur, x[:, 0, :]], axis=0)
    b_enc2 = b_enc.reshape(1, D_SAE)
    b_dec2 = b_dec.reshape(1, D_IN)

    pre = _encode_pre(x2, W_enc, b_enc2)
    thr, nh2 = _topk_threshold(pre)
    x_hat, z_cur, l_hi, l_full = _decode_losses(pre, thr, W_dec, x_cur, b_dec2)
    l_contr = _info_nce_loss(pre, thr, nh2)

    total = (l_hi[0, 0] + l_full[0, 0]) + l_contr[0, 0]
    return (total, x_hat, z_cur)


# speedup vs baseline: 1.0097x; 1.0097x over previous
import jax
import jax.numpy as jnp
from jax import lax
from jax.experimental import pallas as pl
from jax.experimental.pallas import tpu as pltpu

D_IN = 2304
D_SAE = 18432
TOPK = 100
HALF = D_SAE // 2
BATCH = 1024
ROWS2 = 2 * BATCH

ENC_TILE = 768
DEC_TILE = 1024
SIM_TILE = 2304
K2_ROWS = 128
NPASS = 24

_PREC = lax.Precision.DEFAULT


def _k1_body(x_ref, w_ref, b_ref, out_ref):
    out_ref[...] = (
        jnp.dot(x_ref[...], w_ref[...], preferred_element_type=jnp.float32,
                precision=_PREC)
        + b_ref[...]
    )


def _encode_pre(x2, w_enc, b_enc2):
    grid = (D_SAE // ENC_TILE,)
    return pl.pallas_call(
        _k1_body,
        grid=grid,
        in_specs=[
            pl.BlockSpec((ROWS2, D_IN), lambda j: (0, 0)),
            pl.BlockSpec((D_IN, ENC_TILE), lambda j: (0, j)),
            pl.BlockSpec((1, ENC_TILE), lambda j: (0, j)),
        ],
        out_specs=pl.BlockSpec((ROWS2, ENC_TILE), lambda j: (0, j)),
        out_shape=jax.ShapeDtypeStruct((ROWS2, D_SAE), jnp.float32),
    )(x2, w_enc, b_enc2)


def _k2_body(pre_ref, thr_ref, nh2_ref):
    p = pre_ref[...]
    bits = lax.bitcast_convert_type(p, jnp.int32)
    bitsp = jnp.where(p > 0.0, bits, 0)
    k16 = ((bitsp >> 15) - 32768).astype(jnp.int16)
    tu = jnp.zeros((K2_ROWS, 1), jnp.int32)
    c = jnp.zeros((K2_ROWS, 1), jnp.int32)

    def _count16(mask16):
        m16 = mask16.astype(jnp.int16)
        s = m16[:, : D_SAE // 2] + m16[:, D_SAE // 2:]
        s = s[:, : D_SAE // 4] + s[:, D_SAE // 4:]
        s = s[:, : D_SAE // 8] + s[:, D_SAE // 8:]
        s = s[:, : D_SAE // 16] + s[:, D_SAE // 16:]
        return jnp.sum(s.astype(jnp.int32), axis=1, keepdims=True)

    for b in range(15, -1, -1):
        cand = tu + (1 << b)
        cand16 = (cand - 32768).astype(jnp.int16)
        cnt = _count16(k16 >= cand16)
        take = cnt >= TOPK
        tu = jnp.where(take, cand, tu)
        c = jnp.where(take, cnt, c)
    tu16 = (tu - 32768).astype(jnp.int16)
    c_gt = _count16(k16 > tu16)
    lo16 = ((bitsp & 0x7FFF) - 32768).astype(jnp.int16)
    bandlo = jnp.where(k16 == tu16, lo16, jnp.int16(-32768))
    tlo = jnp.zeros((K2_ROWS, 1), jnp.int32)
    for b in range(14, 30 - NPASS, -1):
        cand = tlo + (1 << b)
        cand16 = (cand - 32768).astype(jnp.int16)
        cnt = c_gt + _count16(bandlo >= cand16)
        take = cnt >= TOPK
        tlo = jnp.where(take, cand, tlo)
        c = jnp.where(take, cnt, c)
    t = (tu << 15) + tlo
    thr = jnp.maximum(t, 1)
    c = jnp.where(t > 0, c, 0)

    big = jnp.int32(0x7FFFFFFF)

    def _over(state):
        thr_, c_ = state
        return jnp.any(c_ > TOPK)

    def _drop_min(state):
        thr_, c_ = state
        act = bitsp >= thr_
        mk = jnp.min(jnp.where(act, bitsp, big), axis=1, keepdims=True)
        nmin = jnp.sum((act & (bitsp == mk)).astype(jnp.int32), axis=1,
                       keepdims=True)
        over = c_ > TOPK
        thr_ = jnp.where(over, mk + 1, thr_)
        c_ = jnp.where(over, c_ - nmin, c_)
        return thr_, c_

    thr, c = lax.while_loop(_over, _drop_min, (thr, c))
    thr_ref[...] = thr
    z = jnp.where(bitsp >= thr, p, 0.0)
    col = lax.broadcasted_iota(jnp.int32, (K2_ROWS, D_SAE), 1)
    zh = jnp.where(col < HALF, z, 0.0)
    nh2_ref[...] = jnp.sum(zh * zh, axis=1, keepdims=True)


def _topk_threshold(pre):
    grid = (ROWS2 // K2_ROWS,)
    return pl.pallas_call(
        _k2_body,
        grid=grid,
        in_specs=[pl.BlockSpec((K2_ROWS, D_SAE), lambda i: (i, 0))],
        out_specs=[
            pl.BlockSpec((K2_ROWS, 1), lambda i: (i, 0)),
            pl.BlockSpec((K2_ROWS, 1), lambda i: (i, 0)),
        ],
        out_shape=[
            jax.ShapeDtypeStruct((ROWS2, 1), jnp.int32),
            jax.ShapeDtypeStruct((ROWS2, 1), jnp.float32),
        ],
    )(pre)


def _mask_tile(p, thr):
    bits = lax.bitcast_convert_type(p, jnp.int32)
    return jnp.where(bits >= thr, p, 0.0)


def _k3_body(pre_ref, thr_ref, w_ref, xc_ref, bd_ref,
             xhat_ref, z_ref, lhi_ref, lfull_ref):
    c = pl.program_id(0)
    nhi = HALF // DEC_TILE
    denom = float(BATCH * D_IN)

    @pl.when(c == 0)
    def _init():
        xhat_ref[...] = jnp.zeros_like(xhat_ref)

    zt = _mask_tile(pre_ref[...], thr_ref[...])
    z_ref[...] = zt
    prod = jnp.dot(zt.astype(jnp.bfloat16), w_ref[...].astype(jnp.bfloat16),
                   preferred_element_type=jnp.float32)
    xhat_ref[...] += prod

    @pl.when(c == nhi - 1)
    def _hi():
        dh = (xhat_ref[...] + bd_ref[...]) - xc_ref[...]
        eh = jnp.sum(jnp.sum(dh * dh, axis=1, keepdims=True), axis=0,
                     keepdims=True)
        lhi_ref[...] = eh / denom

    @pl.when(c == (D_SAE // DEC_TILE) - 1)
    def _fin():
        full = xhat_ref[...] + bd_ref[...]
        xhat_ref[...] = full
        d = full - xc_ref[...]
        ef = jnp.sum(jnp.sum(d * d, axis=1, keepdims=True), axis=0,
                     keepdims=True)
        lfull_ref[...] = ef / denom


def _decode_losses(pre, thr, w_dec, x_cur, b_dec2):
    grid = (D_SAE // DEC_TILE,)
    return pl.pallas_call(
        _k3_body,
        grid=grid,
        in_specs=[
            pl.BlockSpec((BATCH, DEC_TILE), lambda c: (0, c)),
            pl.BlockSpec((BATCH, 1), lambda c: (0, 0)),
            pl.BlockSpec((DEC_TILE, D_IN), lambda c: (c, 0)),
            pl.BlockSpec((BATCH, D_IN), lambda c: (0, 0)),
            pl.BlockSpec((1, D_IN), lambda c: (0, 0)),
        ],
        out_specs=[
            pl.BlockSpec((BATCH, D_IN), lambda c: (0, 0)),
            pl.BlockSpec((BATCH, DEC_TILE), lambda c: (0, c)),
            pl.BlockSpec((1, 1), lambda c: (0, 0)),
            pl.BlockSpec((1, 1), lambda c: (0, 0)),
        ],
        out_shape=[
            jax.ShapeDtypeStruct((BATCH, D_IN), jnp.float32),
            jax.ShapeDtypeStruct((BATCH, D_SAE), jnp.float32),
            jax.ShapeDtypeStruct((1, 1), jnp.float32),
            jax.ShapeDtypeStruct((1, 1), jnp.float32),
        ],
    )(pre, thr, w_dec, x_cur, b_dec2)


def _k4_body(pa_ref, pb_ref, ta_ref, tb_ref, na2_ref, nb2_ref, out_ref,
             sim_ref):
    j = pl.program_id(0)

    @pl.when(j == 0)
    def _init():
        sim_ref[...] = jnp.zeros_like(sim_ref)

    na = jnp.maximum(jnp.sqrt(na2_ref[...]), 1e-8)
    nb = jnp.maximum(jnp.sqrt(nb2_ref[...]), 1e-8)
    za = _mask_tile(pa_ref[...], ta_ref[...])
    zb = _mask_tile(pb_ref[...], tb_ref[...])
    zan = (za / na).astype(jnp.bfloat16)
    zbn = (zb / nb).astype(jnp.bfloat16)
    sim_ref[...] += lax.dot_general(
        zan, zbn, (((1,), (1,)), ((), ())),
        preferred_element_type=jnp.float32)

    @pl.when(j == (HALF // SIM_TILE) - 1)
    def _fin():
        s = sim_ref[...]
        rmax = jnp.max(s, axis=1, keepdims=True)
        lse_r = rmax + jnp.log(jnp.sum(jnp.exp(s - rmax), axis=1,
                                       keepdims=True))
        cmax = jnp.max(s, axis=0, keepdims=True)
        lse_c = cmax + jnp.log(jnp.sum(jnp.exp(s - cmax), axis=0,
                                       keepdims=True))
        ii = lax.broadcasted_iota(jnp.int32, (BATCH, BATCH), 0)
        jj = lax.broadcasted_iota(jnp.int32, (BATCH, BATCH), 1)
        dsum = jnp.sum(jnp.sum(jnp.where(ii == jj, s, 0.0), axis=1,
                               keepdims=True), axis=0, keepdims=True)
        sr = jnp.sum(jnp.sum(lse_r, axis=1, keepdims=True), axis=0,
                     keepdims=True)
        sc = jnp.sum(jnp.sum(lse_c, axis=1, keepdims=True), axis=0,
                     keepdims=True)
        out_ref[...] = (-dsum + 0.5 * (sr + sc)) / float(BATCH)


def _info_nce_loss(pre, thr, nh2):
    grid = (HALF // SIM_TILE,)
    return pl.pallas_call(
        _k4_body,
        grid=grid,
        in_specs=[
            pl.BlockSpec((BATCH, SIM_TILE), lambda j: (0, j)),
            pl.BlockSpec((BATCH, SIM_TILE), lambda j: (1, j)),
            pl.BlockSpec((BATCH, 1), lambda j: (0, 0)),
            pl.BlockSpec((BATCH, 1), lambda j: (1, 0)),
            pl.BlockSpec((BATCH, 1), lambda j: (0, 0)),
            pl.BlockSpec((BATCH, 1), lambda j: (1, 0)),
        ],
        out_specs=pl.BlockSpec((1, 1), lambda j: (0, 0)),
        out_shape=jax.ShapeDtypeStruct((1, 1), jnp.float32),
        scratch_shapes=[pltpu.VMEM((BATCH, BATCH), jnp.float32)],
    )(pre, pre, thr, thr, nh2, nh2)


def kernel(x, W_enc, W_dec, b_enc, b_dec):
    x_cur = x[:, 1, :]
    x2 = jnp.concatenate([x_cur, x[:, 0, :]], axis=0)
    b_enc2 = b_enc.reshape(1, D_SAE)
    b_dec2 = b_dec.reshape(1, D_IN)

    pre = _encode_pre(x2, W_enc, b_enc2)
    thr, nh2 = _topk_threshold(pre)
    x_hat, z_cur, l_hi, l_full = _decode_losses(pre, thr, W_dec, x_cur, b_dec2)
    l_contr = _info_nce_loss(pre, thr, nh2)

    total = (l_hi[0, 0] + l_full[0, 0]) + l_contr[0, 0]
    return (total, x_hat, z_cur)
